# Initial kernel scaffold; baseline (speedup 1.0000x reference)
import jax
import jax.numpy as jnp
from jax.experimental import pallas as pl


def kernel(inputs, embedding_weight):
    raise NotImplementedError("write your pallas kernel here")


# speedup vs baseline: 1.2636x; 1.2636x over previous
import jax
import jax.numpy as jnp
from jax.experimental import pallas as pl
from jax.experimental.pallas import tpu as pltpu

NUM_E = 1024
DIM = 64
CC = 0.25
BLK = 512


def _vq_body(x_ref, w_ref, q_ref, idx_ref, loss_ref, acc_ref):
    i = pl.program_id(0)
    xb = x_ref[...]
    w = w_ref[...]
    x2 = jnp.sum(xb * xb, axis=1, keepdims=True)
    dot = jax.lax.dot_general(
        xb, w, (((1,), (1,)), ((), ())),
        preferred_element_type=jnp.float32)
    w2 = jnp.sum(w * w, axis=1)
    d = x2 - 2.0 * dot + w2[None, :]
    dmin = jnp.min(d, axis=1, keepdims=True)
    ks = jax.lax.broadcasted_iota(jnp.int32, d.shape, 1)
    idx = jnp.min(jnp.where(d == dmin, ks, NUM_E), axis=1)
    idx_ref[...] = idx
    onehot = (ks == idx[:, None]).astype(jnp.float32)
    q = jnp.dot(onehot, w, preferred_element_type=jnp.float32)
    q_ref[...] = q
    dq = q - xb
    bs = jnp.sum(dq * dq)

    @pl.when(i == 0)
    def _():
        acc_ref[0] = 0.0

    acc_ref[0] += bs

    @pl.when(i == pl.num_programs(0) - 1)
    def _():
        loss_ref[0, 0] = acc_ref[0]


def kernel(inputs, embedding_weight):
    shape = inputs.shape
    x = inputs.reshape(-1, DIM)
    B = x.shape[0]
    grid = B // BLK
    q, idx, loss = pl.pallas_call(
        _vq_body,
        grid=(grid,),
        in_specs=[
            pl.BlockSpec((BLK, DIM), lambda i: (i, 0)),
            pl.BlockSpec((NUM_E, DIM), lambda i: (0, 0)),
        ],
        out_specs=[
            pl.BlockSpec((BLK, DIM), lambda i: (i, 0)),
            pl.BlockSpec((BLK,), lambda i: (i,)),
            pl.BlockSpec(memory_space=pltpu.SMEM),
        ],
        out_shape=[
            jax.ShapeDtypeStruct((B, DIM), jnp.float32),
            jax.ShapeDtypeStruct((B,), jnp.int32),
            jax.ShapeDtypeStruct((1, 1), jnp.float32),
        ],
        scratch_shapes=[pltpu.SMEM((1,), jnp.float32)],
    )(x, embedding_weight)
    loss = loss[0, 0] * (CC / (B * DIM))
    return q.reshape(shape), loss, idx.reshape(shape[:-1])
